# baseline (speedup 1.0000x reference)
import jax
import jax.numpy as jnp
from jax.experimental import pallas as pl
from jax.experimental.pallas import tpu as pltpu

KW = 256
TT = 128


def _prep_kernel(starts_ref, w_ref, m_ref, b_ref, wsh_ref, bt_ref):
    S, C, W, O = w_ref.shape
    zrows = jnp.zeros((KW - W, O), dtype=jnp.float32)
    bt_ref[...] = b_ref[:, 0, :].transpose(1, 0)

    def body(s, _):
        start = starts_ref[s]
        rem = start % 128
        mask = m_ref[s, 0]
        for c in range(C):
            wm = jnp.concatenate([w_ref[s, c] * mask[:, None], zrows],
                                 axis=0)
            wsh_ref[s, c] = pltpu.roll(wm, rem, axis=0).astype(jnp.bfloat16)
        return 0

    jax.lax.fori_loop(0, S, body, 0, unroll=False)


def _band_kernel(starts_ref, x_ref, wsh_ref, y_ref, xb_ref):
    s = pl.program_id(0)
    B, C, T, F = x_ref.shape
    fbuf = xb_ref.shape[-1]

    @pl.when(s == 0)
    def _cast_x():
        for b in range(B):
            for c in range(C):
                xb_ref[b, c, :, :F] = x_ref[b, c].astype(jnp.bfloat16)
                xb_ref[b, c, :, F:] = jnp.zeros((T, fbuf - F),
                                                dtype=jnp.bfloat16)

    start = starts_ref[s]
    base = (start // 128) * 128
    a0 = xb_ref[:, 0, :, pl.ds(base, KW)].reshape(B * T, KW)
    a1 = xb_ref[:, 1, :, pl.ds(base, KW)].reshape(B * T, KW)
    y = jnp.dot(a0, wsh_ref[s, 0], preferred_element_type=jnp.float32)
    y += jnp.dot(a1, wsh_ref[s, 1], preferred_element_type=jnp.float32)
    y_ref[0] = y.astype(jnp.bfloat16).reshape(B, T, -1)


def _xpose_kernel(y_ref, bt_ref, o_ref):
    o_ref[0] = (y_ref[:, 0].transpose(2, 1, 0).astype(jnp.float32)
                + bt_ref[...][:, None, :])


def kernel(x, pre_w, pre_b, idxes, masks):
    B, C, T, F = x.shape
    S, _, W, O = pre_w.shape
    starts = idxes[:, 0].astype(jnp.int32)
    m_r = masks.reshape(S, 1, W)
    b_r = pre_b.reshape(S, 1, O)
    fbuf = ((F + 127) // 128 + 1) * 128

    prep_spec = pltpu.PrefetchScalarGridSpec(
        num_scalar_prefetch=1,
        grid=(1,),
        in_specs=[
            pl.BlockSpec((S, C, W, O), lambda g, st: (0, 0, 0, 0)),
            pl.BlockSpec((S, 1, W), lambda g, st: (0, 0, 0)),
            pl.BlockSpec((S, 1, O), lambda g, st: (0, 0, 0)),
        ],
        out_specs=[
            pl.BlockSpec((S, C, KW, O), lambda g, st: (0, 0, 0, 0)),
            pl.BlockSpec((O, S), lambda g, st: (0, 0)),
        ],
    )
    wsh, bt = pl.pallas_call(
        _prep_kernel,
        grid_spec=prep_spec,
        out_shape=[
            jax.ShapeDtypeStruct((S, C, KW, O), jnp.bfloat16),
            jax.ShapeDtypeStruct((O, S), jnp.float32),
        ],
    )(starts, pre_w, m_r, b_r)

    band_spec = pltpu.PrefetchScalarGridSpec(
        num_scalar_prefetch=1,
        grid=(S,),
        in_specs=[
            pl.BlockSpec((B, C, T, F), lambda s, st: (0, 0, 0, 0)),
            pl.BlockSpec((S, C, KW, O), lambda s, st: (0, 0, 0, 0)),
        ],
        out_specs=pl.BlockSpec((1, B, T, O), lambda s, st: (s, 0, 0, 0)),
        scratch_shapes=[
            pltpu.VMEM((B, C, T, fbuf), jnp.bfloat16),
        ],
    )
    y = pl.pallas_call(
        _band_kernel,
        grid_spec=band_spec,
        out_shape=jax.ShapeDtypeStruct((S, B, T, O), jnp.bfloat16),
    )(starts, x, wsh)

    out = pl.pallas_call(
        _xpose_kernel,
        grid=(B, T // TT),
        in_specs=[
            pl.BlockSpec((S, 1, TT, O), lambda b, t: (0, b, t, 0)),
            pl.BlockSpec((O, S), lambda b, t: (0, 0)),
        ],
        out_specs=pl.BlockSpec((1, O, TT, S), lambda b, t: (b, 0, t, 0)),
        out_shape=jax.ShapeDtypeStruct((B, O, T, S), jnp.float32),
    )(y, bt)
    return out


# speedup vs baseline: 1.2451x; 1.2451x over previous
import jax
import jax.numpy as jnp
from jax.experimental import pallas as pl
from jax.experimental.pallas import tpu as pltpu

KW = 256
TT = 128


def _prep_kernel(starts_ref, w_ref, m_ref, wsh_ref):
    S, C, W, O = w_ref.shape
    zrows = jnp.zeros((KW - W, O), dtype=jnp.float32)

    def body(s, _):
        start = starts_ref[s]
        rem = start % 128
        mask = m_ref[s, 0]
        for c in range(C):
            wm = jnp.concatenate([w_ref[s, c] * mask[:, None], zrows],
                                 axis=0)
            wsh_ref[s, c] = pltpu.roll(wm, rem, axis=0).astype(jnp.bfloat16)
        return 0

    jax.lax.fori_loop(0, S, body, 0, unroll=False)


def _band_kernel(starts_ref, x_ref, wsh_ref, b_ref, y_ref, xb_ref):
    s = pl.program_id(0)
    B, C, T, F = x_ref.shape
    fbuf = xb_ref.shape[-1]

    @pl.when(s == 0)
    def _cast_x():
        for b in range(B):
            for c in range(C):
                xb_ref[b, c, :, :F] = x_ref[b, c].astype(jnp.bfloat16)
                xb_ref[b, c, :, F:] = jnp.zeros((T, fbuf - F),
                                                dtype=jnp.bfloat16)

    start = starts_ref[s]
    base = (start // 128) * 128
    a0 = xb_ref[:, 0, :, pl.ds(base, KW)].reshape(B * T, KW)
    a1 = xb_ref[:, 1, :, pl.ds(base, KW)].reshape(B * T, KW)
    y = jnp.dot(a0, wsh_ref[s, 0], preferred_element_type=jnp.float32)
    y += jnp.dot(a1, wsh_ref[s, 1], preferred_element_type=jnp.float32)
    y += b_ref[s, 0][None, :]
    y_ref[0] = y.astype(jnp.bfloat16).reshape(B, T, -1)


def kernel(x, pre_w, pre_b, idxes, masks):
    B, C, T, F = x.shape
    S, _, W, O = pre_w.shape
    starts = idxes[:, 0].astype(jnp.int32)
    m_r = masks.reshape(S, 1, W)
    b_r = pre_b.reshape(S, 1, O)
    fbuf = ((F + 127) // 128 + 1) * 128

    prep_spec = pltpu.PrefetchScalarGridSpec(
        num_scalar_prefetch=1,
        grid=(1,),
        in_specs=[
            pl.BlockSpec((S, C, W, O), lambda g, st: (0, 0, 0, 0)),
            pl.BlockSpec((S, 1, W), lambda g, st: (0, 0, 0)),
        ],
        out_specs=pl.BlockSpec((S, C, KW, O), lambda g, st: (0, 0, 0, 0)),
    )
    wsh = pl.pallas_call(
        _prep_kernel,
        grid_spec=prep_spec,
        out_shape=jax.ShapeDtypeStruct((S, C, KW, O), jnp.bfloat16),
    )(starts, pre_w, m_r)

    band_spec = pltpu.PrefetchScalarGridSpec(
        num_scalar_prefetch=1,
        grid=(S,),
        in_specs=[
            pl.BlockSpec((B, C, T, F), lambda s, st: (0, 0, 0, 0)),
            pl.BlockSpec((S, C, KW, O), lambda s, st: (0, 0, 0, 0)),
            pl.BlockSpec((S, 1, O), lambda s, st: (0, 0, 0)),
        ],
        out_specs=pl.BlockSpec((1, B, T, O), lambda s, st: (s, 0, 0, 0)),
        scratch_shapes=[
            pltpu.VMEM((B, C, T, fbuf), jnp.bfloat16),
        ],
    )
    y = pl.pallas_call(
        _band_kernel,
        grid_spec=band_spec,
        out_shape=jax.ShapeDtypeStruct((S, B, T, O), jnp.bfloat16),
    )(starts, x, wsh, b_r)

    return y.transpose(1, 3, 2, 0).astype(jnp.float32)
